# baseline (speedup 1.0000x reference)
import functools

import jax
import jax.numpy as jnp
from jax import lax
from jax.experimental import pallas as pl
from jax.experimental.pallas import tpu as pltpu
from jax.experimental.pallas import tpu_sc as plsc

NC = 2
NS = 16
L = 16
C = 512


def _make_sc_segsum(K, rows_per_tile, Dh):
  N_pad = NS * rows_per_tile
  mesh = plsc.VectorSubcoreMesh(core_axis_name="c", subcore_axis_name="s",
                                num_cores=NC)

  @functools.partial(
      pl.kernel,
      out_type=(
          jax.ShapeDtypeStruct((NC, N_pad, Dh), jnp.float32),
          jax.ShapeDtypeStruct((1, 1, N_pad), jnp.float32),
      ),
      mesh=mesh,
      compiler_params=pltpu.CompilerParams(use_tc_tiling_on_sc=False),
      scratch_types=[
          pltpu.VMEM((K, C), jnp.int32),
          pltpu.VMEM((K, C), jnp.int32),
          pltpu.VMEM((C, Dh), jnp.float32),
          pltpu.VMEM((C,), jnp.float32),
          pltpu.VMEM((128, Dh), jnp.float32),
          pltpu.VMEM((rows_per_tile,), jnp.float32),
          pltpu.VMEM_SHARED((N_pad, Dh), jnp.float32),
          pltpu.VMEM_SHARED((N_pad,), jnp.float32),
          pltpu.SemaphoreType.DMA,
      ],
  )
  def sc_segsum(x_hbm, src_hbm, dst_hbm, z2_hbm, z1_hbm,
                s_out, cnt_out,
                src_v, dst_v, rows_v, ones_v, z2_v, z1_v, acc, cacc, sem):
    cid = lax.axis_index("c")
    sid = lax.axis_index("s")
    base = sid * rows_per_tile

    pltpu.sync_copy(src_hbm.at[cid, sid], src_v)
    pltpu.sync_copy(dst_hbm.at[sid], dst_v)
    pltpu.sync_copy(z2_hbm, z2_v)
    pltpu.sync_copy(z1_hbm, z1_v)

    @pl.loop(0, C // L)
    def _(t):
      ones_v[pl.ds(t * L, L)] = jnp.full((L,), 1.0, jnp.float32)

    full, rem = divmod(rows_per_tile, 128)
    for i in range(full):
      pltpu.sync_copy(z2_v, acc.at[pl.ds(base + i * 128, 128)])
    if rem:
      pltpu.sync_copy(z2_v.at[pl.ds(0, rem)],
                      acc.at[pl.ds(base + full * 128, rem)])
    pltpu.sync_copy(z1_v, cacc.at[pl.ds(base, rows_per_tile)])
    plsc.subcore_barrier()

    @pl.loop(0, K)
    def _(g):
      pltpu.async_copy(x_hbm.at[src_v.at[g]], rows_v, sem).wait()
      pltpu.sync_copy(rows_v, acc.at[dst_v.at[g]], add=True)

      @pl.when(cid == 0)
      def _():
        pltpu.sync_copy(ones_v, cacc.at[dst_v.at[g]], add=True)

    plsc.subcore_barrier()
    pltpu.sync_copy(acc.at[pl.ds(base, rows_per_tile)],
                    s_out.at[cid, pl.ds(base, rows_per_tile)])

    @pl.when(cid == 0)
    def _():
      pltpu.sync_copy(cacc.at[pl.ds(base, rows_per_tile)],
                      cnt_out.at[0, 0, pl.ds(base, rows_per_tile)])

  return sc_segsum


def _make_tc_combine(N, N_pad, D, H, relu):
  def body(s_ref, c_ref, x_ref, wl_ref, bl_ref, wr_ref, o_ref):
    s = jnp.concatenate([s_ref[0], s_ref[1]], axis=-1)
    c = c_ref[0]
    mean = s * (1.0 / jnp.maximum(c, 1.0))
    out = (
        lax.dot_general(mean[:N], wl_ref[...],
                        (((1,), (1,)), ((), ())),
                        preferred_element_type=jnp.float32)
        + bl_ref[...][None, :]
        + lax.dot_general(x_ref[...], wr_ref[...],
                          (((1,), (1,)), ((), ())),
                          preferred_element_type=jnp.float32)
    )
    o_ref[...] = jnp.maximum(out, 0.0) if relu else out

  return pl.pallas_call(
      body,
      out_shape=jax.ShapeDtypeStruct((N, H), jnp.float32),
  )


def _prep_edges(edge_index, N, K):
  E = edge_index.shape[1]
  E_pad = NS * K * C
  src = jnp.concatenate(
      [edge_index[0],
       jnp.zeros((E_pad - E,), jnp.int32)]).reshape(NS, K, C)
  src = jnp.stack([src, src + N])
  dst = jnp.concatenate(
      [edge_index[1],
       jnp.full((E_pad - E,), N, jnp.int32)]).reshape(NS, K, C)
  return lax.optimization_barrier((src, dst))


def kernel(x, edge_index, W1l, b1l, W1r, W2l, b2l, W2r):
  N, D = x.shape
  H = W1l.shape[0]
  O = W2l.shape[0]
  E = edge_index.shape[1]
  Dh = D // NC

  K = -(-E // (NS * C))
  rows_per_tile = -(-(N + 1) // (NS * 128)) * 128
  N_pad = NS * rows_per_tile

  src, dst = _prep_edges(edge_index, N, K)
  z2 = jnp.zeros((128, Dh), jnp.float32)
  z1 = jnp.zeros((rows_per_tile,), jnp.float32)

  sc_segsum = _make_sc_segsum(K, rows_per_tile, Dh)
  tc1 = _make_tc_combine(N, N_pad, D, H, relu=True)
  tc2 = _make_tc_combine(N, N_pad, H, O, relu=False)

  def split(v):
    return lax.optimization_barrier(
        jnp.concatenate([v[:, :Dh], v[:, Dh:]], axis=0))

  s1, cnt = sc_segsum(split(x), src, dst, z2, z1)
  cnt = cnt.reshape(1, N_pad, 1)
  h = tc1(s1, cnt, x, W1l, b1l, W1r)
  s2, _ = sc_segsum(split(h), src, dst, z2, z1)
  out = tc2(s2, cnt, h, W2l, b2l, W2r)
  return out


# speedup vs baseline: 1.4282x; 1.4282x over previous
import functools

import jax
import jax.numpy as jnp
from jax import lax
from jax.experimental import pallas as pl
from jax.experimental.pallas import tpu as pltpu
from jax.experimental.pallas import tpu_sc as plsc

NC = 2
NS = 16
L = 16
C = 256


def _make_sc_segsum(K, rows_per_tile, Dh):
  N_pad = NS * rows_per_tile
  mesh = plsc.VectorSubcoreMesh(core_axis_name="c", subcore_axis_name="s",
                                num_cores=NC)

  @functools.partial(
      pl.kernel,
      out_type=(
          jax.ShapeDtypeStruct((NC, N_pad, Dh), jnp.float32),
          jax.ShapeDtypeStruct((1, 1, N_pad), jnp.float32),
      ),
      mesh=mesh,
      compiler_params=pltpu.CompilerParams(use_tc_tiling_on_sc=False),
      scratch_types=[
          pltpu.VMEM((2, 2, C), jnp.int32),
          pltpu.VMEM((C, Dh), jnp.float32),
          pltpu.VMEM((C,), jnp.float32),
          pltpu.VMEM((128, Dh), jnp.float32),
          pltpu.VMEM((rows_per_tile,), jnp.float32),
          pltpu.VMEM_SHARED((N_pad, Dh), jnp.float32),
          pltpu.VMEM_SHARED((N_pad, Dh), jnp.float32),
          pltpu.VMEM_SHARED((N_pad,), jnp.float32),
          pltpu.SemaphoreType.DMA,
          pltpu.SemaphoreType.DMA((2,)),
      ],
  )
  def sc_segsum(x_hbm, eib_hbm, z2_hbm, z1_hbm,
                s_out, cnt_out,
                idx_v, rows_v, ones_v, z2_v, z1_v, table, acc, cacc,
                sem, isems):
    cid = lax.axis_index("c")
    sid = lax.axis_index("s")
    base = sid * rows_per_tile

    pltpu.sync_copy(z2_hbm, z2_v)
    pltpu.sync_copy(z1_hbm, z1_v)
    for b in range(2):
      pltpu.async_copy(eib_hbm.at[sid, b], idx_v.at[b], isems.at[b])
    pltpu.sync_copy(x_hbm.at[pl.ds(cid * N_pad + base, rows_per_tile)],
                    table.at[pl.ds(base, rows_per_tile)])

    @pl.loop(0, C // L)
    def _(t):
      ones_v[pl.ds(t * L, L)] = jnp.full((L,), 1.0, jnp.float32)

    full, rem = divmod(rows_per_tile, 128)
    for i in range(full):
      pltpu.sync_copy(z2_v, acc.at[pl.ds(base + i * 128, 128)])
    if rem:
      pltpu.sync_copy(z2_v.at[pl.ds(0, rem)],
                      acc.at[pl.ds(base + full * 128, rem)])
    pltpu.sync_copy(z1_v, cacc.at[pl.ds(base, rows_per_tile)])
    plsc.subcore_barrier()

    @pl.loop(0, K // 2)
    def _(g):
      for b in range(2):
        j = g * 2 + b
        pltpu.make_async_copy(eib_hbm.at[sid, j], idx_v.at[b],
                              isems.at[b]).wait()
        pltpu.sync_copy(table.at[idx_v.at[b, 0]], rows_v)
        pltpu.sync_copy(rows_v, acc.at[idx_v.at[b, 1]], add=True)

        @pl.when(cid == 0)
        def _():
          pltpu.sync_copy(ones_v, cacc.at[idx_v.at[b, 1]], add=True)

        @pl.when(j + 2 < K)
        def _():
          pltpu.async_copy(eib_hbm.at[sid, j + 2], idx_v.at[b], isems.at[b])

    plsc.subcore_barrier()
    pltpu.sync_copy(acc.at[pl.ds(base, rows_per_tile)],
                    s_out.at[cid, pl.ds(base, rows_per_tile)])

    @pl.when(cid == 0)
    def _():
      pltpu.sync_copy(cacc.at[pl.ds(base, rows_per_tile)],
                      cnt_out.at[0, 0, pl.ds(base, rows_per_tile)])

  return sc_segsum


def _make_tc_combine(N, N_pad, D, H, relu):
  def body(s_ref, c_ref, x_ref, wl_ref, bl_ref, wr_ref, o_ref):
    s = jnp.concatenate([s_ref[0], s_ref[1]], axis=-1)
    c = c_ref[0]
    mean = s * (1.0 / jnp.maximum(c, 1.0))
    out = (
        lax.dot_general(mean[:N], wl_ref[...],
                        (((1,), (1,)), ((), ())),
                        preferred_element_type=jnp.float32)
        + bl_ref[...][None, :]
        + lax.dot_general(x_ref[...], wr_ref[...],
                          (((1,), (1,)), ((), ())),
                          preferred_element_type=jnp.float32)
    )
    o_ref[...] = jnp.maximum(out, 0.0) if relu else out

  return pl.pallas_call(
      body,
      out_shape=jax.ShapeDtypeStruct((N, H), jnp.float32),
  )


def _prep_edges(edge_index, N, K):
  E = edge_index.shape[1]
  E_pad = NS * K * C
  src = jnp.concatenate(
      [edge_index[0], jnp.zeros((E_pad - E,), jnp.int32)]).reshape(NS, K, C)
  dst = jnp.concatenate(
      [edge_index[1],
       jnp.full((E_pad - E,), N, jnp.int32)]).reshape(NS, K, C)
  return lax.optimization_barrier(jnp.stack([src, dst], axis=2))


def kernel(x, edge_index, W1l, b1l, W1r, W2l, b2l, W2r):
  N, D = x.shape
  H = W1l.shape[0]
  O = W2l.shape[0]
  E = edge_index.shape[1]
  Dh = D // NC

  K = -(-E // (NS * C * 2)) * 2
  rows_per_tile = -(-(N + 1) // (NS * 128)) * 128
  N_pad = NS * rows_per_tile

  eib = _prep_edges(edge_index, N, K)
  z2 = jnp.zeros((128, Dh), jnp.float32)
  z1 = jnp.zeros((rows_per_tile,), jnp.float32)

  sc_segsum = _make_sc_segsum(K, rows_per_tile, Dh)
  tc1 = _make_tc_combine(N, N_pad, D, H, relu=True)
  tc2 = _make_tc_combine(N, N_pad, H, O, relu=False)

  def split(v):
    vp = jnp.pad(v, ((0, N_pad - N), (0, 0)))
    return lax.optimization_barrier(
        jnp.concatenate([vp[:, :Dh], vp[:, Dh:]], axis=0))

  s1, cnt = sc_segsum(split(x), eib, z2, z1)
  cnt = cnt.reshape(1, N_pad, 1)
  h = tc1(s1, cnt, x, W1l, b1l, W1r)
  s2, _ = sc_segsum(split(h), eib, z2, z1)
  out = tc2(s2, cnt, h, W2l, b2l, W2r)
  return out


# speedup vs baseline: 1.6355x; 1.1451x over previous
import functools

import jax
import jax.numpy as jnp
from jax import lax
from jax.experimental import pallas as pl
from jax.experimental.pallas import tpu as pltpu
from jax.experimental.pallas import tpu_sc as plsc

NC = 2
NS = 16
L = 16
C = 256


def _make_sc_segsum(K, rows_per_tile, Dh):
  N_pad = NS * rows_per_tile
  mesh = plsc.VectorSubcoreMesh(core_axis_name="c", subcore_axis_name="s",
                                num_cores=NC)

  @functools.partial(
      pl.kernel,
      out_type=(
          jax.ShapeDtypeStruct((NC, N_pad, Dh), jnp.float32),
          jax.ShapeDtypeStruct((1, 1, N_pad), jnp.float32),
      ),
      mesh=mesh,
      compiler_params=pltpu.CompilerParams(use_tc_tiling_on_sc=False),
      scratch_types=[
          pltpu.VMEM((2, 2, C), jnp.int32),
          pltpu.VMEM((2, C, Dh), jnp.float32),
          pltpu.VMEM((C,), jnp.float32),
          pltpu.VMEM((128, Dh), jnp.float32),
          pltpu.VMEM((rows_per_tile,), jnp.float32),
          pltpu.VMEM_SHARED((N_pad, Dh), jnp.float32),
          pltpu.VMEM_SHARED((N_pad, Dh), jnp.float32),
          pltpu.VMEM_SHARED((N_pad,), jnp.float32),
          pltpu.SemaphoreType.DMA((2,)),
          pltpu.SemaphoreType.DMA((2,)),
      ],
  )
  def sc_segsum(x_hbm, eib_hbm, z2_hbm, z1_hbm,
                s_out, cnt_out,
                idx_v, rows_v, ones_v, z2_v, z1_v, table, acc, cacc,
                gsems, isems):
    cid = lax.axis_index("c")
    sid = lax.axis_index("s")
    base = sid * rows_per_tile

    pltpu.sync_copy(z2_hbm, z2_v)
    pltpu.sync_copy(z1_hbm, z1_v)
    for b in range(2):
      pltpu.async_copy(eib_hbm.at[sid, b], idx_v.at[b], isems.at[b])
    pltpu.sync_copy(x_hbm.at[pl.ds(cid * N_pad + base, rows_per_tile)],
                    table.at[pl.ds(base, rows_per_tile)])

    @pl.loop(0, C // L)
    def _(t):
      ones_v[pl.ds(t * L, L)] = jnp.full((L,), 1.0, jnp.float32)

    full, rem = divmod(rows_per_tile, 128)
    for i in range(full):
      pltpu.sync_copy(z2_v, acc.at[pl.ds(base + i * 128, 128)])
    if rem:
      pltpu.sync_copy(z2_v.at[pl.ds(0, rem)],
                      acc.at[pl.ds(base + full * 128, rem)])
    pltpu.sync_copy(z1_v, cacc.at[pl.ds(base, rows_per_tile)])
    plsc.subcore_barrier()

    pltpu.make_async_copy(eib_hbm.at[sid, 0], idx_v.at[0], isems.at[0]).wait()
    pltpu.async_copy(table.at[idx_v.at[0, 0]], rows_v.at[0], gsems.at[0])

    @pl.loop(0, K // 2)
    def _(g):
      for b in range(2):
        j = g * 2 + b
        bn = 1 - b
        pltpu.make_async_copy(table.at[idx_v.at[b, 0]], rows_v.at[b],
                              gsems.at[b]).wait()

        @pl.when(j + 1 < K)
        def _():
          pltpu.make_async_copy(eib_hbm.at[sid, j + 1], idx_v.at[bn],
                                isems.at[bn]).wait()
          pltpu.async_copy(table.at[idx_v.at[bn, 0]], rows_v.at[bn],
                           gsems.at[bn])

        pltpu.sync_copy(rows_v.at[b], acc.at[idx_v.at[b, 1]], add=True)

        @pl.when(cid == 0)
        def _():
          pltpu.sync_copy(ones_v, cacc.at[idx_v.at[b, 1]], add=True)

        @pl.when(j + 2 < K)
        def _():
          pltpu.async_copy(eib_hbm.at[sid, j + 2], idx_v.at[b], isems.at[b])

    plsc.subcore_barrier()
    pltpu.sync_copy(acc.at[pl.ds(base, rows_per_tile)],
                    s_out.at[cid, pl.ds(base, rows_per_tile)])

    @pl.when(cid == 0)
    def _():
      pltpu.sync_copy(cacc.at[pl.ds(base, rows_per_tile)],
                      cnt_out.at[0, 0, pl.ds(base, rows_per_tile)])

  return sc_segsum


def _make_tc_combine(N, N_pad, D, H, relu):
  def body(s_ref, c_ref, x_ref, wl_ref, bl_ref, wr_ref, o_ref):
    s = jnp.concatenate([s_ref[0], s_ref[1]], axis=-1)
    c = c_ref[0]
    mean = s * (1.0 / jnp.maximum(c, 1.0))
    out = (
        lax.dot_general(mean[:N], wl_ref[...],
                        (((1,), (1,)), ((), ())),
                        preferred_element_type=jnp.float32)
        + bl_ref[...][None, :]
        + lax.dot_general(x_ref[...], wr_ref[...],
                          (((1,), (1,)), ((), ())),
                          preferred_element_type=jnp.float32)
    )
    o_ref[...] = jnp.maximum(out, 0.0) if relu else out

  return pl.pallas_call(
      body,
      out_shape=jax.ShapeDtypeStruct((N, H), jnp.float32),
  )


def _prep_edges(edge_index, N, K):
  E = edge_index.shape[1]
  E_pad = NS * K * C
  src = jnp.concatenate(
      [edge_index[0], jnp.zeros((E_pad - E,), jnp.int32)]).reshape(NS, K, C)
  dst = jnp.concatenate(
      [edge_index[1],
       jnp.full((E_pad - E,), N, jnp.int32)]).reshape(NS, K, C)
  return lax.optimization_barrier(jnp.stack([src, dst], axis=2))


def kernel(x, edge_index, W1l, b1l, W1r, W2l, b2l, W2r):
  N, D = x.shape
  H = W1l.shape[0]
  O = W2l.shape[0]
  E = edge_index.shape[1]
  Dh = D // NC

  K = -(-E // (NS * C * 2)) * 2
  rows_per_tile = -(-(N + 1) // (NS * 128)) * 128
  N_pad = NS * rows_per_tile

  eib = _prep_edges(edge_index, N, K)
  z2 = jnp.zeros((128, Dh), jnp.float32)
  z1 = jnp.zeros((rows_per_tile,), jnp.float32)

  sc_segsum = _make_sc_segsum(K, rows_per_tile, Dh)
  tc1 = _make_tc_combine(N, N_pad, D, H, relu=True)
  tc2 = _make_tc_combine(N, N_pad, H, O, relu=False)

  def split(v):
    vp = jnp.pad(v, ((0, N_pad - N), (0, 0)))
    return lax.optimization_barrier(
        jnp.concatenate([vp[:, :Dh], vp[:, Dh:]], axis=0))

  s1, cnt = sc_segsum(split(x), eib, z2, z1)
  cnt = cnt.reshape(1, N_pad, 1)
  h = tc1(s1, cnt, x, W1l, b1l, W1r)
  s2, _ = sc_segsum(split(h), eib, z2, z1)
  out = tc2(s2, cnt, h, W2l, b2l, W2r)
  return out


# speedup vs baseline: 1.6542x; 1.0114x over previous
import functools

import jax
import jax.numpy as jnp
from jax import lax
from jax.experimental import pallas as pl
from jax.experimental.pallas import tpu as pltpu
from jax.experimental.pallas import tpu_sc as plsc

NC = 2
NS = 16
L = 16
C = 256


def _make_sc_segsum(K, rows_per_tile, Dh):
  N_pad = NS * rows_per_tile
  mesh = plsc.VectorSubcoreMesh(core_axis_name="c", subcore_axis_name="s",
                                num_cores=NC)

  @functools.partial(
      pl.kernel,
      out_type=(
          jax.ShapeDtypeStruct((NC, N_pad, Dh), jnp.float32),
          jax.ShapeDtypeStruct((1, 1, N_pad), jnp.float32),
      ),
      mesh=mesh,
      compiler_params=pltpu.CompilerParams(use_tc_tiling_on_sc=False),
      scratch_types=[
          pltpu.VMEM((2, 2, C), jnp.int32),
          pltpu.VMEM((2, C, Dh), jnp.float32),
          pltpu.VMEM((C,), jnp.float32),
          pltpu.VMEM((128, Dh), jnp.float32),
          pltpu.VMEM((rows_per_tile,), jnp.float32),
          pltpu.VMEM_SHARED((N_pad, Dh), jnp.float32),
          pltpu.VMEM_SHARED((N_pad, Dh), jnp.float32),
          pltpu.VMEM_SHARED((N_pad,), jnp.float32),
          pltpu.SemaphoreType.DMA((2,)),
          pltpu.SemaphoreType.DMA((2,)),
      ],
  )
  def sc_segsum(x_hbm, eib_hbm, z2_hbm, z1_hbm,
                s_out, cnt_out,
                idx_v, rows_v, ones_v, z2_v, z1_v, table, acc, cacc,
                gsems, isems):
    cid = lax.axis_index("c")
    sid = lax.axis_index("s")
    base = sid * rows_per_tile

    pltpu.sync_copy(z2_hbm, z2_v)
    pltpu.sync_copy(z1_hbm, z1_v)
    for b in range(2):
      pltpu.async_copy(eib_hbm.at[sid, b], idx_v.at[b], isems.at[b])
    pltpu.sync_copy(x_hbm.at[pl.ds(cid * N_pad + base, rows_per_tile)],
                    table.at[pl.ds(base, rows_per_tile)])

    @pl.loop(0, C // L)
    def _(t):
      ones_v[pl.ds(t * L, L)] = jnp.full((L,), 1.0, jnp.float32)

    full, rem = divmod(rows_per_tile, 128)
    for i in range(full):
      pltpu.sync_copy(z2_v, acc.at[pl.ds(base + i * 128, 128)])
    if rem:
      pltpu.sync_copy(z2_v.at[pl.ds(0, rem)],
                      acc.at[pl.ds(base + full * 128, rem)])
    pltpu.sync_copy(z1_v, cacc.at[pl.ds(base, rows_per_tile)])
    plsc.subcore_barrier()

    pltpu.make_async_copy(eib_hbm.at[sid, 0], idx_v.at[0], isems.at[0]).wait()
    pltpu.async_copy(table.at[idx_v.at[0, 0]], rows_v.at[0], gsems.at[0])

    @pl.loop(0, K // 2)
    def _(g):
      for b in range(2):
        j = g * 2 + b
        bn = 1 - b
        pltpu.make_async_copy(table.at[idx_v.at[b, 0]], rows_v.at[b],
                              gsems.at[b]).wait()

        @pl.when(j + 1 < K)
        def _():
          pltpu.make_async_copy(eib_hbm.at[sid, j + 1], idx_v.at[bn],
                                isems.at[bn]).wait()
          pltpu.async_copy(table.at[idx_v.at[bn, 0]], rows_v.at[bn],
                           gsems.at[bn])

        pltpu.sync_copy(rows_v.at[b], acc.at[idx_v.at[b, 1]], add=True)

        @pl.when(cid == 0)
        def _():
          pltpu.sync_copy(ones_v, cacc.at[idx_v.at[b, 1]], add=True)

        @pl.when(j + 2 < K)
        def _():
          pltpu.async_copy(eib_hbm.at[sid, j + 2], idx_v.at[b], isems.at[b])

    plsc.subcore_barrier()
    pltpu.sync_copy(acc.at[pl.ds(base, rows_per_tile)],
                    s_out.at[cid, pl.ds(base, rows_per_tile)])

    @pl.when(cid == 0)
    def _():
      pltpu.sync_copy(cacc.at[pl.ds(base, rows_per_tile)],
                      cnt_out.at[0, 0, pl.ds(base, rows_per_tile)])

  return sc_segsum


def _make_tc_combine(N, N_pad, D, H, relu, split_in, split_out):
  Hh = H // NC

  def body(s_ref, c_ref, x_ref, wl_ref, bl_ref, wr_ref, o_ref):
    s = jnp.concatenate([s_ref[0], s_ref[1]], axis=-1)
    c = c_ref[0]
    mean = s * (1.0 / jnp.maximum(c, 1.0))
    if split_in:
      xx = jnp.concatenate([x_ref[pl.ds(0, N)], x_ref[pl.ds(N_pad, N)]],
                           axis=-1)
    else:
      xx = x_ref[...]
    out = (
        lax.dot_general(mean[:N], wl_ref[...],
                        (((1,), (1,)), ((), ())),
                        preferred_element_type=jnp.float32)
        + bl_ref[...][None, :]
        + lax.dot_general(xx, wr_ref[...],
                          (((1,), (1,)), ((), ())),
                          preferred_element_type=jnp.float32)
    )
    out = jnp.maximum(out, 0.0) if relu else out
    if split_out:
      o_ref[pl.ds(0, N), :] = out[:, :Hh]
      o_ref[pl.ds(N_pad, N), :] = out[:, Hh:]
    else:
      o_ref[...] = out

  out_shape = ((NC * N_pad, Hh) if split_out else (N, H))
  return pl.pallas_call(
      body,
      out_shape=jax.ShapeDtypeStruct(out_shape, jnp.float32),
  )


def _prep_edges(edge_index, N, K):
  E = edge_index.shape[1]
  E_pad = NS * K * C
  src = jnp.concatenate(
      [edge_index[0], jnp.zeros((E_pad - E,), jnp.int32)]).reshape(NS, K, C)
  dst = jnp.concatenate(
      [edge_index[1],
       jnp.full((E_pad - E,), N, jnp.int32)]).reshape(NS, K, C)
  return lax.optimization_barrier(jnp.stack([src, dst], axis=2))


def kernel(x, edge_index, W1l, b1l, W1r, W2l, b2l, W2r):
  N, D = x.shape
  H = W1l.shape[0]
  O = W2l.shape[0]
  E = edge_index.shape[1]
  Dh = D // NC

  K = -(-E // (NS * C * 2)) * 2
  rows_per_tile = -(-(N + 1) // (NS * 128)) * 128
  N_pad = NS * rows_per_tile

  eib = _prep_edges(edge_index, N, K)
  z2 = jnp.zeros((128, Dh), jnp.float32)
  z1 = jnp.zeros((rows_per_tile,), jnp.float32)

  sc_segsum = _make_sc_segsum(K, rows_per_tile, Dh)
  tc1 = _make_tc_combine(N, N_pad, D, H, relu=True,
                         split_in=False, split_out=True)
  tc2 = _make_tc_combine(N, N_pad, H, O, relu=False,
                         split_in=True, split_out=False)

  def split(v):
    vp = jnp.pad(v, ((0, N_pad - N), (0, 0)))
    return lax.optimization_barrier(
        jnp.concatenate([vp[:, :Dh], vp[:, Dh:]], axis=0))

  s1, cnt = sc_segsum(split(x), eib, z2, z1)
  cnt = cnt.reshape(1, N_pad, 1)
  hs = tc1(s1, cnt, x, W1l, b1l, W1r)
  s2, _ = sc_segsum(hs, eib, z2, z1)
  out = tc2(s2, cnt, hs, W2l, b2l, W2r)
  return out


# speedup vs baseline: 1.6672x; 1.0079x over previous
import functools

import jax
import jax.numpy as jnp
from jax import lax
from jax.experimental import pallas as pl
from jax.experimental.pallas import tpu as pltpu
from jax.experimental.pallas import tpu_sc as plsc

NC = 2
NS = 16
L = 16
C = 256


def _make_sc_segsum(K, rows_per_tile, Dh, with_cnt):
  N_pad = NS * rows_per_tile
  mesh = plsc.VectorSubcoreMesh(core_axis_name="c", subcore_axis_name="s",
                                num_cores=NC)
  out_type = [jax.ShapeDtypeStruct((NC, N_pad, Dh), jnp.float32)]
  if with_cnt:
    out_type.append(jax.ShapeDtypeStruct((1, 1, N_pad), jnp.float32))

  @functools.partial(
      pl.kernel,
      out_type=tuple(out_type),
      mesh=mesh,
      compiler_params=pltpu.CompilerParams(use_tc_tiling_on_sc=False),
      scratch_types=[
          pltpu.VMEM((2, 2, C), jnp.int32),
          pltpu.VMEM((2, C, Dh), jnp.float32),
          pltpu.VMEM((C,), jnp.float32),
          pltpu.VMEM((128, Dh), jnp.float32),
          pltpu.VMEM((rows_per_tile,), jnp.float32),
          pltpu.VMEM_SHARED((N_pad, Dh), jnp.float32),
          pltpu.VMEM_SHARED((N_pad, Dh), jnp.float32),
          pltpu.VMEM_SHARED((N_pad,), jnp.float32),
          pltpu.SemaphoreType.DMA((2,)),
          pltpu.SemaphoreType.DMA((2,)),
      ],
  )
  def sc_segsum(x_hbm, eib_hbm, z2_hbm, z1_hbm,
                s_out, *rest):
    if with_cnt:
      cnt_out, idx_v, rows_v, ones_v, z2_v, z1_v, table, acc, cacc, \
          gsems, isems = rest
    else:
      idx_v, rows_v, ones_v, z2_v, z1_v, table, acc, cacc, \
          gsems, isems = rest
    cid = lax.axis_index("c")
    sid = lax.axis_index("s")
    base = sid * rows_per_tile

    pltpu.sync_copy(z2_hbm, z2_v)
    pltpu.sync_copy(z1_hbm, z1_v)
    for b in range(2):
      pltpu.async_copy(eib_hbm.at[sid, b], idx_v.at[b], isems.at[b])
    pltpu.sync_copy(x_hbm.at[pl.ds(cid * N_pad + base, rows_per_tile)],
                    table.at[pl.ds(base, rows_per_tile)])

    if with_cnt:
      @pl.loop(0, C // L)
      def _(t):
        ones_v[pl.ds(t * L, L)] = jnp.full((L,), 1.0, jnp.float32)

    full, rem = divmod(rows_per_tile, 128)
    for i in range(full):
      pltpu.sync_copy(z2_v, acc.at[pl.ds(base + i * 128, 128)])
    if rem:
      pltpu.sync_copy(z2_v.at[pl.ds(0, rem)],
                      acc.at[pl.ds(base + full * 128, rem)])
    if with_cnt:
      pltpu.sync_copy(z1_v, cacc.at[pl.ds(base, rows_per_tile)])
    plsc.subcore_barrier()

    pltpu.make_async_copy(eib_hbm.at[sid, 0], idx_v.at[0], isems.at[0]).wait()
    pltpu.async_copy(table.at[idx_v.at[0, 0]], rows_v.at[0], gsems.at[0])

    @pl.loop(0, K // 2)
    def _(g):
      for b in range(2):
        j = g * 2 + b
        bn = 1 - b
        pltpu.make_async_copy(table.at[idx_v.at[b, 0]], rows_v.at[b],
                              gsems.at[b]).wait()

        @pl.when(j + 1 < K)
        def _():
          pltpu.make_async_copy(eib_hbm.at[sid, j + 1], idx_v.at[bn],
                                isems.at[bn]).wait()
          pltpu.async_copy(table.at[idx_v.at[bn, 0]], rows_v.at[bn],
                           gsems.at[bn])

        pltpu.sync_copy(rows_v.at[b], acc.at[idx_v.at[b, 1]], add=True)

        if with_cnt:
          @pl.when(cid == 0)
          def _():
            pltpu.sync_copy(ones_v, cacc.at[idx_v.at[b, 1]], add=True)

        @pl.when(j + 2 < K)
        def _():
          pltpu.async_copy(eib_hbm.at[sid, j + 2], idx_v.at[b], isems.at[b])

    plsc.subcore_barrier()
    pltpu.sync_copy(acc.at[pl.ds(base, rows_per_tile)],
                    s_out.at[cid, pl.ds(base, rows_per_tile)])

    if with_cnt:
      @pl.when(cid == 0)
      def _():
        pltpu.sync_copy(cacc.at[pl.ds(base, rows_per_tile)],
                        cnt_out.at[0, 0, pl.ds(base, rows_per_tile)])

  return sc_segsum


def _make_tc_combine(N, N_pad, D, H, relu, split_in, split_out):
  Hh = H // NC

  def body(s_ref, c_ref, x_ref, wl_ref, bl_ref, wr_ref, o_ref):
    s = jnp.concatenate([s_ref[0], s_ref[1]], axis=-1)
    c = c_ref[0]
    mean = s * (1.0 / jnp.maximum(c, 1.0))
    if split_in:
      xx = jnp.concatenate([x_ref[pl.ds(0, N)], x_ref[pl.ds(N_pad, N)]],
                           axis=-1)
    else:
      xx = x_ref[...]
    out = (
        lax.dot_general(mean[:N], wl_ref[...],
                        (((1,), (1,)), ((), ())),
                        preferred_element_type=jnp.float32)
        + bl_ref[...][None, :]
        + lax.dot_general(xx, wr_ref[...],
                          (((1,), (1,)), ((), ())),
                          preferred_element_type=jnp.float32)
    )
    out = jnp.maximum(out, 0.0) if relu else out
    if split_out:
      o_ref[pl.ds(0, N), :] = out[:, :Hh]
      o_ref[pl.ds(N_pad, N), :] = out[:, Hh:]
    else:
      o_ref[...] = out

  out_shape = ((NC * N_pad, Hh) if split_out else (N, H))
  return pl.pallas_call(
      body,
      out_shape=jax.ShapeDtypeStruct(out_shape, jnp.float32),
  )


def _prep_edges(edge_index, N, K):
  E = edge_index.shape[1]
  E_pad = NS * K * C
  src = jnp.concatenate(
      [edge_index[0], jnp.zeros((E_pad - E,), jnp.int32)]).reshape(NS, K, C)
  dst = jnp.concatenate(
      [edge_index[1],
       jnp.full((E_pad - E,), N, jnp.int32)]).reshape(NS, K, C)
  return lax.optimization_barrier(jnp.stack([src, dst], axis=2))


def kernel(x, edge_index, W1l, b1l, W1r, W2l, b2l, W2r):
  N, D = x.shape
  H = W1l.shape[0]
  O = W2l.shape[0]
  E = edge_index.shape[1]
  Dh = D // NC

  K = -(-E // (NS * C * 2)) * 2
  rows_per_tile = -(-(N + 1) // (NS * 128)) * 128
  N_pad = NS * rows_per_tile

  eib = _prep_edges(edge_index, N, K)
  z2 = jnp.zeros((128, Dh), jnp.float32)
  z1 = jnp.zeros((rows_per_tile,), jnp.float32)

  sc_segsum = _make_sc_segsum(K, rows_per_tile, Dh, with_cnt=True)
  sc_segsum2 = _make_sc_segsum(K, rows_per_tile, Dh, with_cnt=False)
  tc1 = _make_tc_combine(N, N_pad, D, H, relu=True,
                         split_in=False, split_out=True)
  tc2 = _make_tc_combine(N, N_pad, H, O, relu=False,
                         split_in=True, split_out=False)

  def split(v):
    vp = jnp.pad(v, ((0, N_pad - N), (0, 0)))
    return lax.optimization_barrier(
        jnp.concatenate([vp[:, :Dh], vp[:, Dh:]], axis=0))

  s1, cnt = sc_segsum(split(x), eib, z2, z1)
  cnt = cnt.reshape(1, N_pad, 1)
  hs = tc1(s1, cnt, x, W1l, b1l, W1r)
  (s2,) = sc_segsum2(hs, eib, z2, z1)
  out = tc2(s2, cnt, hs, W2l, b2l, W2r)
  return out
